# baseline (speedup 1.0000x reference)
import jax
import jax.numpy as jnp
from jax.experimental import pallas as pl

_BLK = 1000


def _body(s_ref, n_ref, ws_ref, wn_ref, bs_ref, bn_ref, o_ref):
    inv_s = 1.0 / s_ref.shape[1]
    inv_n = 1.0 / n_ref.shape[1]
    smean = jnp.sum(s_ref[...], axis=1) * inv_s
    nmean = jnp.sum(n_ref[...], axis=1) * inv_n
    y = jnp.dot(smean, ws_ref[...], preferred_element_type=jnp.float32)
    y = y + jnp.dot(nmean, wn_ref[...], preferred_element_type=jnp.float32)
    o_ref[...] = jnp.maximum(y + (bs_ref[...] + bn_ref[...]), 0.0)


def kernel(self_vecs, neigh_vecs, W_neigh, b_neigh, W_self, b_self):
    n_nodes, s_self, d = self_vecs.shape
    s_neigh = neigh_vecs.shape[1]
    blk = _BLK
    grid = (n_nodes // blk,)

    return pl.pallas_call(
        _body,
        grid=grid,
        in_specs=[
            pl.BlockSpec((blk, s_self, d), lambda i: (i, 0, 0)),
            pl.BlockSpec((blk, s_neigh, d), lambda i: (i, 0, 0)),
            pl.BlockSpec((d, d), lambda i: (0, 0)),
            pl.BlockSpec((d, d), lambda i: (0, 0)),
            pl.BlockSpec((1, d), lambda i: (0, 0)),
            pl.BlockSpec((1, d), lambda i: (0, 0)),
        ],
        out_specs=pl.BlockSpec((blk, d), lambda i: (i, 0)),
        out_shape=jax.ShapeDtypeStruct((n_nodes, d), jnp.float32),
    )(self_vecs, neigh_vecs, W_self, W_neigh,
      b_self.reshape(1, d), b_neigh.reshape(1, d))


# speedup vs baseline: 1.0061x; 1.0061x over previous
import jax
import jax.numpy as jnp
from jax.experimental import pallas as pl

_BLK = 400


def _body(s_ref, n_ref, ws_ref, wn_ref, bs_ref, bn_ref, o_ref):
    inv_s = 1.0 / s_ref.shape[1]
    inv_n = 1.0 / n_ref.shape[1]
    smean = jnp.sum(s_ref[...], axis=1) * inv_s
    nmean = jnp.sum(n_ref[...], axis=1) * inv_n
    y = jnp.dot(smean, ws_ref[...], preferred_element_type=jnp.float32)
    y = y + jnp.dot(nmean, wn_ref[...], preferred_element_type=jnp.float32)
    o_ref[...] = jnp.maximum(y + (bs_ref[...] + bn_ref[...]), 0.0)


def kernel(self_vecs, neigh_vecs, W_neigh, b_neigh, W_self, b_self):
    n_nodes, s_self, d = self_vecs.shape
    s_neigh = neigh_vecs.shape[1]
    blk = _BLK
    grid = (n_nodes // blk,)

    return pl.pallas_call(
        _body,
        grid=grid,
        in_specs=[
            pl.BlockSpec((blk, s_self, d), lambda i: (i, 0, 0)),
            pl.BlockSpec((blk, s_neigh, d), lambda i: (i, 0, 0)),
            pl.BlockSpec((d, d), lambda i: (0, 0)),
            pl.BlockSpec((d, d), lambda i: (0, 0)),
            pl.BlockSpec((1, d), lambda i: (0, 0)),
            pl.BlockSpec((1, d), lambda i: (0, 0)),
        ],
        out_specs=pl.BlockSpec((blk, d), lambda i: (i, 0)),
        out_shape=jax.ShapeDtypeStruct((n_nodes, d), jnp.float32),
    )(self_vecs, neigh_vecs, W_self, W_neigh,
      b_self.reshape(1, d), b_neigh.reshape(1, d))
